# Initial kernel scaffold; baseline (speedup 1.0000x reference)
import jax
import jax.numpy as jnp
from jax.experimental import pallas as pl


def kernel(logits, neighborhood_temps):
    raise NotImplementedError("write your pallas kernel here")


# speedup vs baseline: 199.3275x; 199.3275x over previous
import jax
import jax.numpy as jnp
from jax import lax
from jax.experimental import pallas as pl
from jax.experimental.pallas import tpu as pltpu

_EPS = 1e-12
_TH = 128
_HB = 8


def _nectar_kernel(temps_ref, x_ref, top_ref, bot_ref, o_ref):
    h = pl.program_id(1)
    nh = pl.num_programs(1)
    x = x_ref[0]
    _, th, w = x.shape

    pred = jnp.argmax(x, axis=0).astype(jnp.int32)
    top_row = jnp.argmax(top_ref[0, :, _HB - 1:_HB, :], axis=0).astype(jnp.int32)
    bot_row = jnp.argmax(bot_ref[0, :, 0:1, :], axis=0).astype(jnp.int32)
    top_row = jnp.where(h == 0, -1, top_row)
    bot_row = jnp.where(h == nh - 1, -1, bot_row)

    padded = jnp.concatenate([top_row, pred, bot_row], axis=0)
    col = lax.broadcasted_iota(jnp.int32, (th, w), 1)

    count = jnp.zeros((th, w), dtype=jnp.int32)
    for di in (-1, 0, 1):
        rows = padded[1 + di:1 + di + th, :]
        for dj in (-1, 0, 1):
            if di == 0 and dj == 0:
                continue
            if dj == 0:
                nb = rows
            else:
                nb = jnp.roll(rows, -dj, axis=1)
                edge = w - 1 if dj == 1 else 0
                nb = jnp.where(col == edge, -1, nb)
            count = count + (nb == pred).astype(jnp.int32)

    temp = jnp.full((th, w), temps_ref[0], dtype=jnp.float32)
    for k in range(1, 9):
        temp = jnp.where(count == k, temps_ref[k], temp)

    t = jnp.maximum(temp, 0.0) + _EPS
    o_ref[0] = x / t[None, :, :]


def kernel(logits, neighborhood_temps):
    B, C, H, W = logits.shape
    nh = H // _TH
    nhb = _TH // _HB
    return pl.pallas_call(
        _nectar_kernel,
        grid=(B, nh),
        in_specs=[
            pl.BlockSpec(memory_space=pltpu.SMEM),
            pl.BlockSpec((1, C, _TH, W), lambda b, h: (b, 0, h, 0)),
            pl.BlockSpec((1, C, _HB, W),
                         lambda b, h: (b, 0, jnp.maximum(h * nhb - 1, 0), 0)),
            pl.BlockSpec((1, C, _HB, W),
                         lambda b, h: (b, 0, jnp.minimum((h + 1) * nhb, H // _HB - 1), 0)),
        ],
        out_specs=pl.BlockSpec((1, C, _TH, W), lambda b, h: (b, 0, h, 0)),
        out_shape=jax.ShapeDtypeStruct(logits.shape, logits.dtype),
    )(neighborhood_temps, logits, logits, logits)


# speedup vs baseline: 207.8170x; 1.0426x over previous
import jax
import jax.numpy as jnp
from jax import lax
from jax.experimental import pallas as pl
from jax.experimental.pallas import tpu as pltpu

_EPS = 1e-12
_TH = 256
_HB = 8


def _nectar_kernel(temps_ref, x_ref, top_ref, bot_ref, o_ref):
    h = pl.program_id(1)
    nh = pl.num_programs(1)
    x = x_ref[0]
    _, th, w = x.shape

    pred = jnp.argmax(x, axis=0).astype(jnp.int32)
    top_row = jnp.argmax(top_ref[0, :, _HB - 1:_HB, :], axis=0).astype(jnp.int32)
    bot_row = jnp.argmax(bot_ref[0, :, 0:1, :], axis=0).astype(jnp.int32)
    top_row = jnp.where(h == 0, -1, top_row)
    bot_row = jnp.where(h == nh - 1, -1, bot_row)

    padded = jnp.concatenate([top_row, pred, bot_row], axis=0)
    col = lax.broadcasted_iota(jnp.int32, (th, w), 1)

    count = jnp.zeros((th, w), dtype=jnp.int32)
    for di in (-1, 0, 1):
        rows = padded[1 + di:1 + di + th, :]
        for dj in (-1, 0, 1):
            if di == 0 and dj == 0:
                continue
            if dj == 0:
                nb = rows
            else:
                nb = jnp.roll(rows, -dj, axis=1)
                edge = w - 1 if dj == 1 else 0
                nb = jnp.where(col == edge, -1, nb)
            count = count + (nb == pred).astype(jnp.int32)

    temp = jnp.full((th, w), temps_ref[0], dtype=jnp.float32)
    for k in range(1, 9):
        temp = jnp.where(count == k, temps_ref[k], temp)

    t = jnp.maximum(temp, 0.0) + _EPS
    o_ref[0] = x / t[None, :, :]


def kernel(logits, neighborhood_temps):
    B, C, H, W = logits.shape
    nh = H // _TH
    nhb = _TH // _HB
    return pl.pallas_call(
        _nectar_kernel,
        grid=(B, nh),
        in_specs=[
            pl.BlockSpec(memory_space=pltpu.SMEM),
            pl.BlockSpec((1, C, _TH, W), lambda b, h: (b, 0, h, 0)),
            pl.BlockSpec((1, C, _HB, W),
                         lambda b, h: (b, 0, jnp.maximum(h * nhb - 1, 0), 0)),
            pl.BlockSpec((1, C, _HB, W),
                         lambda b, h: (b, 0, jnp.minimum((h + 1) * nhb, H // _HB - 1), 0)),
        ],
        out_specs=pl.BlockSpec((1, C, _TH, W), lambda b, h: (b, 0, h, 0)),
        out_shape=jax.ShapeDtypeStruct(logits.shape, logits.dtype),
    )(neighborhood_temps, logits, logits, logits)


# speedup vs baseline: 210.5049x; 1.0129x over previous
import jax
import jax.numpy as jnp
from jax import lax
from jax.experimental import pallas as pl
from jax.experimental.pallas import tpu as pltpu

_EPS = 1e-12
_TH = 256
_HB = 8


def _body(temps_ref, x_ref, bot_ref, o_ref, carry_ref):
    h = pl.program_id(1)
    nh = pl.num_programs(1)
    x = x_ref[0]
    _, th, w = x.shape

    pred = jnp.argmax(x, axis=0).astype(jnp.int32)
    top_row = jnp.where(h == 0, -1, carry_ref[0:1, :])
    bot_row = jnp.argmax(bot_ref[0, :, 0:1, :], axis=0).astype(jnp.int32)
    bot_row = jnp.where(h == nh - 1, -1, bot_row)
    carry_ref[0:1, :] = pred[th - 1:th, :]

    padded = jnp.concatenate([top_row, pred, bot_row], axis=0)
    col = lax.broadcasted_iota(jnp.int32, (th, w), 1)

    count = jnp.zeros((th, w), dtype=jnp.int32)
    for di in (-1, 0, 1):
        rows = padded[1 + di:1 + di + th, :]
        for dj in (-1, 0, 1):
            if di == 0 and dj == 0:
                continue
            if dj == 0:
                nb = rows
            else:
                nb = jnp.roll(rows, -dj, axis=1)
                edge = w - 1 if dj == 1 else 0
                nb = jnp.where(col == edge, -1, nb)
            count = count + (nb == pred).astype(jnp.int32)

    temp = jnp.full((th, w), temps_ref[0], dtype=jnp.float32)
    for k in range(1, 9):
        temp = jnp.where(count == k, temps_ref[k], temp)

    t = jnp.maximum(temp, 0.0) + _EPS
    o_ref[0] = x / t[None, :, :]


def kernel(logits, neighborhood_temps):
    B, C, H, W = logits.shape
    th = min(_TH, H)
    nh = H // th
    nhb = th // _HB
    return pl.pallas_call(
        _body,
        grid=(B, nh),
        in_specs=[
            pl.BlockSpec(memory_space=pltpu.SMEM),
            pl.BlockSpec((1, C, th, W), lambda b, h: (b, 0, h, 0)),
            pl.BlockSpec((1, C, _HB, W),
                         lambda b, h: (b, 0, jnp.minimum((h + 1) * nhb, H // _HB - 1), 0)),
        ],
        out_specs=pl.BlockSpec((1, C, th, W), lambda b, h: (b, 0, h, 0)),
        out_shape=jax.ShapeDtypeStruct(logits.shape, logits.dtype),
        scratch_shapes=[pltpu.VMEM((8, W), jnp.int32)],
    )(neighborhood_temps, logits, logits)
